# baseline (speedup 1.0000x reference)
import functools

import jax
import jax.numpy as jnp
from jax import lax
from jax.experimental import pallas as pl
from jax.experimental.pallas import tpu as pltpu
from jax.experimental.pallas import tpu_sc as plsc

B = 8
N = 2048
SEGS = 2
SEG = N // SEGS
V_PAD = 1024
L = 16
CHUNKS = SEG // L


def _body(x_hbm, out_hbm, xv, ov, hist, nb0, spm, sem):
    s = lax.axis_index("s")
    row = s // SEGS
    seg = s % SEGS
    base = row * N + seg * SEG

    in_cp = pltpu.async_copy(x_hbm.at[pl.ds(base, SEG)], xv, sem)

    def zero(i, _):
        hist[pl.ds(i * L, L)] = jnp.zeros((L,), jnp.float32)
        return 0

    lax.fori_loop(0, V_PAD // L, zero, 0)
    in_cp.wait()

    def chunk(i, _):
        v = xv[pl.ds(i * L, L)]
        prev = plsc.load_gather(hist, [v])
        rank, last = plsc.scan_count(v)
        cnt = prev + rank.astype(jnp.float32)
        ov[pl.ds(i * L, L)] = cnt
        plsc.store_scatter(hist, [v], cnt, mask=last)
        return 0

    lax.fori_loop(0, CHUNKS, chunk, 0)

    @pl.when(seg == 0)
    def _():
        out_cp = pltpu.async_copy(ov, out_hbm.at[pl.ds(base, SEG)], sem)
        pltpu.sync_copy(hist, spm.at[s])
        out_cp.wait()

    plsc.subcore_barrier()

    @pl.when(seg > 0)
    def _():
        pltpu.sync_copy(spm.at[s - 1], nb0)

        def off(i, _):
            d = pl.ds(i * L, L)
            ov[d] = ov[d] + plsc.load_gather(nb0, [xv[d]])
            return 0

        lax.fori_loop(0, CHUNKS // 2, off, 0)
        h1 = pltpu.async_copy(
            ov.at[pl.ds(0, SEG // 2)], out_hbm.at[pl.ds(base, SEG // 2)], sem
        )
        lax.fori_loop(CHUNKS // 2, CHUNKS, off, 0)
        h2 = pltpu.async_copy(
            ov.at[pl.ds(SEG // 2, SEG // 2)],
            out_hbm.at[pl.ds(base + SEG // 2, SEG // 2)],
            sem,
        )
        h1.wait()
        h2.wait()


@jax.jit
def _counts(x):
    run = pl.kernel(
        _body,
        out_type=jax.ShapeDtypeStruct((B * N,), jnp.float32),
        mesh=plsc.VectorSubcoreMesh(
            core_axis_name="c", subcore_axis_name="s", num_cores=1
        ),
        scratch_types=[
            pltpu.VMEM((SEG,), jnp.int32),
            pltpu.VMEM((SEG,), jnp.float32),
            pltpu.VMEM((V_PAD,), jnp.float32),
            pltpu.VMEM((V_PAD,), jnp.float32),
            pltpu.VMEM_SHARED((16, V_PAD), jnp.float32),
            pltpu.SemaphoreType.DMA,
        ],
        compiler_params=pltpu.CompilerParams(
            needs_layout_passes=False, use_tc_tiling_on_sc=False
        ),
    )
    return run(x.astype(jnp.int32).reshape(B * N))


def kernel(x):
    return _counts(x).reshape(B, N, 1)


# speedup vs baseline: 1.0036x; 1.0036x over previous
import functools

import jax
import jax.numpy as jnp
from jax import lax
from jax.experimental import pallas as pl
from jax.experimental.pallas import tpu as pltpu
from jax.experimental.pallas import tpu_sc as plsc

B = 8
N = 2048
SEGS = 2
SEG = N // SEGS
V_PAD = 1024
L = 16
CHUNKS = SEG // L


def _body(x_hbm, out_hbm, xv, ov, hist, nb0, spm, sem):
    s = lax.axis_index("s")
    row = s // SEGS
    seg = s % SEGS
    base = row * N + seg * SEG

    in_cp = pltpu.async_copy(x_hbm.at[pl.ds(base, SEG)], xv, sem)

    def zero(i, _):
        hist[pl.ds(i * L, L)] = jnp.zeros((L,), jnp.float32)
        return 0

    lax.fori_loop(0, V_PAD // L, zero, 0)
    in_cp.wait()

    def chunk(i, _):
        v = xv[pl.ds(i * L, L)]
        prev = plsc.load_gather(hist, [v])
        rank, last = plsc.scan_count(v)
        cnt = prev + rank.astype(jnp.float32)
        ov[pl.ds(i * L, L)] = cnt
        plsc.store_scatter(hist, [v], cnt, mask=last)
        return 0

    lax.fori_loop(0, CHUNKS, chunk, 0)

    @pl.when(seg == 0)
    def _():
        pltpu.sync_copy(hist, spm.at[s])

    plsc.subcore_barrier()

    @pl.when(seg > 0)
    def _():
        pltpu.sync_copy(spm.at[s - 1], nb0)

        def off(i, _):
            d = pl.ds(i * L, L)
            ov[d] = ov[d] + plsc.load_gather(nb0, [xv[d]])
            return 0

        lax.fori_loop(0, CHUNKS, off, 0)

    pltpu.sync_copy(ov, out_hbm.at[pl.ds(base, SEG)])


@jax.jit
def _counts(x):
    run = pl.kernel(
        _body,
        out_type=jax.ShapeDtypeStruct((B * N,), jnp.float32),
        mesh=plsc.VectorSubcoreMesh(
            core_axis_name="c", subcore_axis_name="s", num_cores=1
        ),
        scratch_types=[
            pltpu.VMEM((SEG,), jnp.int32),
            pltpu.VMEM((SEG,), jnp.float32),
            pltpu.VMEM((V_PAD,), jnp.float32),
            pltpu.VMEM((V_PAD,), jnp.float32),
            pltpu.VMEM_SHARED((16, V_PAD), jnp.float32),
            pltpu.SemaphoreType.DMA,
        ],
        compiler_params=pltpu.CompilerParams(
            needs_layout_passes=False, use_tc_tiling_on_sc=False
        ),
    )
    return run(x.astype(jnp.int32).reshape(B * N))


def kernel(x):
    return _counts(x).reshape(B, N, 1)
